# Initial kernel scaffold; baseline (speedup 1.0000x reference)
import functools

import jax
import jax.numpy as jnp
from jax import lax
from jax.experimental import pallas as pl
from jax.experimental.pallas import tpu as pltpu
from jax.experimental.pallas import tpu_sc as plsc

NUM_EMB = 1000000
D = 64
B = 4096
S = 50

NC = 2
NS = 16
NW = NC * NS

CB = 2
CHUNK_IDX = CB * S + 4
B_PER_W = B // NW
CHUNKS_PER_W = B_PER_W // CB
IDX_PER_W = CHUNKS_PER_W * CHUNK_IDX


@functools.partial(
    pl.kernel,
    mesh=plsc.VectorSubcoreMesh(core_axis_name="c", subcore_axis_name="s"),
    out_type=jax.ShapeDtypeStruct((B, D), jnp.float32),
    scratch_types=[
        pltpu.VMEM((IDX_PER_W,), jnp.int32),
        pltpu.VMEM((CHUNK_IDX, D), jnp.float32),
        pltpu.VMEM((B_PER_W, D), jnp.float32),
        pltpu.SemaphoreType.DMA,
    ],
)
def _cbow_sc(idx_hbm, table_hbm, out_hbm, idx_v, rows_v, out_v, sem):
    wid = lax.axis_index("s") * NC + lax.axis_index("c")

    pltpu.sync_copy(idx_hbm.at[pl.ds(wid * IDX_PER_W, IDX_PER_W)], idx_v)

    inv_s = jnp.float32(1.0 / S)

    def chunk_body(c, carry):
        off = pl.multiple_of(c * CHUNK_IDX, 8)
        idx_sl = idx_v.at[pl.ds(off, CHUNK_IDX)]
        pltpu.async_copy(table_hbm.at[idx_sl], rows_v, sem).wait()

        for b in range(CB):
            acc = [jnp.zeros((16,), jnp.float32) for _ in range(4)]
            for r in range(S):
                row_i = b * S + r
                s_idx = idx_v[off + row_i]
                m = jnp.where(s_idx == 0, jnp.float32(0.0), jnp.float32(1.0))
                for q in range(4):
                    v = rows_v[row_i, pl.ds(q * 16, 16)]
                    acc[q] = acc[q] + v * m
            for q in range(4):
                out_v[c * CB + b, pl.ds(q * 16, 16)] = acc[q] * inv_s
        return carry

    lax.fori_loop(0, CHUNKS_PER_W, chunk_body, None)

    pltpu.sync_copy(out_v, out_hbm.at[pl.ds(wid * B_PER_W, B_PER_W)])


def kernel(indices, table):
    idx = indices.astype(jnp.int32).reshape(B // CB, CB * S)
    idx = jnp.pad(idx, ((0, 0), (0, CHUNK_IDX - CB * S)))
    return _cbow_sc(idx.reshape(-1), table)


# speedup vs baseline: 1.1754x; 1.1754x over previous
import functools

import jax
import jax.numpy as jnp
from jax import lax
from jax.experimental import pallas as pl
from jax.experimental.pallas import tpu as pltpu
from jax.experimental.pallas import tpu_sc as plsc

D = 64
B = 4096
S = 50

NC = 2
NS = 16
NW = NC * NS

CB = 2
ROWS = CB * S
CHUNK_IDX = 112
B_PER_W = B // NW
CHUNKS_PER_W = B_PER_W // CB
IDX_PER_W = CHUNKS_PER_W * CHUNK_IDX


@functools.partial(
    pl.kernel,
    mesh=plsc.VectorSubcoreMesh(core_axis_name="c", subcore_axis_name="s"),
    out_type=jax.ShapeDtypeStruct((B, D), jnp.float32),
    compiler_params=pltpu.CompilerParams(
        needs_layout_passes=False, use_tc_tiling_on_sc=False
    ),
    scratch_types=[
        pltpu.VMEM((IDX_PER_W,), jnp.int32),
        pltpu.VMEM((ROWS, D), jnp.float32),
        pltpu.VMEM((1, D), jnp.float32),
        pltpu.VMEM((B_PER_W, D), jnp.float32),
        pltpu.SemaphoreType.DMA,
    ],
)
def _cbow_sc(idx_hbm, table_hbm, out_hbm, idx_v, rows_v, t0_v, out_v, sem):
    wid = lax.axis_index("s") * NC + lax.axis_index("c")

    pltpu.sync_copy(idx_hbm.at[pl.ds(wid * IDX_PER_W, IDX_PER_W)], idx_v)
    pltpu.sync_copy(table_hbm.at[pl.ds(0, 1)], t0_v)

    t0 = [t0_v[0, pl.ds(q * 16, 16)] for q in range(4)]
    lane = lax.iota(jnp.int32, 16)
    inv_s = jnp.float32(1.0 / S)

    def popcnt(zb):
        return plsc.all_reduce_population_count(zb)

    def chunk_body(c, carry):
        off = pl.multiple_of(c * CHUNK_IDX, 8)
        idx_sl = idx_v.at[pl.ds(off, ROWS)]
        pltpu.async_copy(table_hbm.at[idx_sl], rows_v, sem).wait()

        z = []
        for v in range(7):
            iv = idx_v[pl.ds(off + v * 16, 16)]
            z.append(iv == 0)
        cnt0 = (
            popcnt(z[0])
            + popcnt(z[1])
            + popcnt(z[2])
            + popcnt(jnp.logical_and(z[3], lane < 2))
        ).astype(jnp.float32)
        cnt1 = (
            popcnt(jnp.logical_and(z[3], lane >= 2))
            + popcnt(z[4])
            + popcnt(z[5])
            + popcnt(jnp.logical_and(z[6], lane < 4))
        ).astype(jnp.float32)

        for b in range(CB):
            acc = [jnp.zeros((16,), jnp.float32) for _ in range(4)]
            for r in range(S):
                row_i = b * S + r
                for q in range(4):
                    acc[q] = acc[q] + rows_v[row_i, pl.ds(q * 16, 16)]
            cnt = cnt0 if b == 0 else cnt1
            for q in range(4):
                out_v[c * CB + b, pl.ds(q * 16, 16)] = (
                    acc[q] - t0[q] * cnt
                ) * inv_s
        return carry

    lax.fori_loop(0, CHUNKS_PER_W, chunk_body, None)

    pltpu.sync_copy(out_v, out_hbm.at[pl.ds(wid * B_PER_W, B_PER_W)])


def kernel(indices, table):
    idx = indices.astype(jnp.int32).reshape(B // CB, ROWS)
    idx = jnp.pad(idx, ((0, 0), (0, CHUNK_IDX - ROWS)))
    return _cbow_sc(idx.reshape(-1), table)
